# baseline (speedup 1.0000x reference)
import functools

import jax
import jax.numpy as jnp
from jax import lax
from jax.experimental import pallas as pl
from jax.experimental.pallas import tpu as pltpu
from jax.experimental.pallas import tpu_sc as plsc

_NC = 2
_NS = 16
_NW = _NC * _NS
_CH = 128
_PW = 32
_NQ = 4
_DREP = 256


def _sc_mesh():
  return plsc.VectorSubcoreMesh(core_axis_name="c", subcore_axis_name="s",
                                num_cores=_NC, num_subcores=_NS)


_NB = 4
_LK = 2


def _ring_seg(table_hbm, src_v, dst_v, acc, bufs, gsems, ssems, k):
  for j in range(_LK):
    pltpu.async_copy(table_hbm.at[src_v.at[j]], bufs[j], gsems[j])

  def body(g, carry):
    for b in range(_NB):
      j = g * _NB + b
      pltpu.make_async_copy(table_hbm.at[src_v.at[j]], bufs[b],
                            gsems[b]).wait()
      pltpu.async_copy(bufs[b], acc.at[dst_v.at[j]], ssems[b], add=True)
      jp = j + _LK
      bp = (b + _LK) % _NB

      @pl.when(jp < k)
      def _():
        @pl.when(jp >= _NB)
        def _():
          pltpu.make_async_copy(bufs[bp], acc.at[dst_v.at[jp - _NB]],
                                ssems[bp]).wait()
        pltpu.async_copy(table_hbm.at[src_v.at[jp]], bufs[bp], gsems[bp])
    return carry

  lax.fori_loop(0, k // _NB, body, 0)
  for b in range(_NB):
    pltpu.make_async_copy(bufs[b], acc.at[dst_v.at[k - _NB + b]],
                          ssems[b]).wait()


def _seg_sum_call(n_pad, k_per_tile):
  grp = n_pad // _NS
  rows = _NS * k_per_tile

  @functools.partial(
      pl.kernel,
      out_type=jax.ShapeDtypeStruct((_NQ * n_pad, _PW), jnp.float32),
      mesh=_sc_mesh(),
      compiler_params=pltpu.CompilerParams(use_tc_tiling_on_sc=False),
      scratch_types=(
          [pltpu.VMEM((k_per_tile, _CH), jnp.int32),
           pltpu.VMEM((k_per_tile, _CH), jnp.int32)]
          + [pltpu.VMEM((_CH, _PW), jnp.float32)] * _NB
          + [pltpu.VMEM_SHARED((n_pad, _PW), jnp.float32)]
          + [pltpu.SemaphoreType.DMA] * (2 * _NB)
      ),
  )
  def k(src_hbm, dst_hbm, table_hbm, zeros_hbm, out_hbm,
        src_v, dst_v, *rest):
    bufs = rest[:_NB]
    acc = rest[_NB]
    gsems = rest[_NB + 1:2 * _NB + 1]
    ssems = rest[2 * _NB + 1:]
    c = lax.axis_index("c")
    s = lax.axis_index("s")
    pltpu.sync_copy(dst_hbm.at[pl.ds(s * k_per_tile, k_per_tile)], dst_v)
    for p in range(2):
      q = 2 * c + p
      pltpu.sync_copy(zeros_hbm.at[pl.ds(s * grp, grp)],
                      acc.at[pl.ds(s * grp, grp)])
      pltpu.sync_copy(
          src_hbm.at[pl.ds(q * rows + s * k_per_tile, k_per_tile)], src_v)
      plsc.subcore_barrier()
      _ring_seg(table_hbm, src_v, dst_v, acc, bufs, gsems, ssems,
                k_per_tile)
      plsc.subcore_barrier()
      pltpu.sync_copy(acc.at[pl.ds(s * grp, grp)],
                      out_hbm.at[pl.ds(q * n_pad + s * grp, grp)])

  return k


def _degree_call(n_pad, k_per_tile):
  drows = n_pad // 16
  grp = drows // _NS

  @functools.partial(
      pl.kernel,
      out_type=jax.ShapeDtypeStruct((_NC * drows, 16), jnp.float32),
      mesh=_sc_mesh(),
      compiler_params=pltpu.CompilerParams(use_tc_tiling_on_sc=False),
      scratch_types=(
          [pltpu.VMEM((k_per_tile, _CH), jnp.int32),
           pltpu.VMEM((k_per_tile, _CH), jnp.int32)]
          + [pltpu.VMEM((_CH, 16), jnp.float32)] * _NB
          + [pltpu.VMEM_SHARED((drows, 16), jnp.float32)]
          + [pltpu.SemaphoreType.DMA] * (2 * _NB)
      ),
  )
  def k(lane_hbm, drow_hbm, eye_hbm, zeros_hbm, out_hbm,
        lane_v, drow_v, *rest):
    bufs = rest[:_NB]
    acc = rest[_NB]
    gsems = rest[_NB + 1:2 * _NB + 1]
    ssems = rest[2 * _NB + 1:]
    c = lax.axis_index("c")
    s = lax.axis_index("s")
    wid = s * _NC + c
    pltpu.sync_copy(zeros_hbm.at[pl.ds(s * grp, grp)],
                    acc.at[pl.ds(s * grp, grp)])
    base = wid * k_per_tile
    pltpu.sync_copy(lane_hbm.at[pl.ds(base, k_per_tile)], lane_v)
    pltpu.sync_copy(drow_hbm.at[pl.ds(base, k_per_tile)], drow_v)
    plsc.subcore_barrier()
    _ring_seg(eye_hbm, lane_v, drow_v, acc, bufs, gsems, ssems, k_per_tile)
    plsc.subcore_barrier()
    pltpu.sync_copy(acc.at[pl.ds(s * grp, grp)],
                    out_hbm.at[pl.ds(c * drows + s * grp, grp)])

  return k


_ENB = 4


def _edge_gather_call(n_pad, e_pad):
  rows_half = e_pad // _CH
  k_w = rows_half // _NS
  lk = _ENB // 2

  @functools.partial(
      pl.kernel,
      out_type=jax.ShapeDtypeStruct((2 * e_pad, 128), jnp.float32),
      mesh=_sc_mesh(),
      compiler_params=pltpu.CompilerParams(use_tc_tiling_on_sc=False),
      scratch_types=(
          [pltpu.VMEM((k_w, _CH), jnp.int32)]
          + [pltpu.VMEM((_CH, 128), jnp.float32)] * _ENB
          + [pltpu.SemaphoreType.DMA] * (2 * _ENB)
      ),
  )
  def k(emb_hbm, idx_hbm, out_hbm, idx_v, *rest):
    bufs = rest[:_ENB]
    gsems = rest[_ENB:2 * _ENB]
    wsems = rest[2 * _ENB:]
    c = lax.axis_index("c")
    s = lax.axis_index("s")
    wid = s * _NC + c
    t = wid % 2
    widx = wid // 2
    pltpu.sync_copy(
        idx_hbm.at[pl.ds(t * rows_half + widx * k_w, k_w)], idx_v)
    obase = t * e_pad + widx * k_w * _CH

    for j in range(lk):
      pltpu.async_copy(emb_hbm.at[idx_v.at[j]], bufs[j], gsems[j])

    def body(g, carry):
      for b in range(_ENB):
        j = g * _ENB + b
        pltpu.make_async_copy(emb_hbm.at[idx_v.at[j]], bufs[b],
                              gsems[b]).wait()
        pltpu.async_copy(bufs[b], out_hbm.at[pl.ds(obase + j * _CH, _CH)],
                         wsems[b])
        jp = j + lk
        bp = (b + lk) % _ENB

        @pl.when(jp < k_w)
        def _():
          @pl.when(jp >= _ENB)
          def _():
            pltpu.make_async_copy(
                bufs[bp],
                out_hbm.at[pl.ds(obase + (jp - _ENB) * _CH, _CH)],
                wsems[bp]).wait()
          pltpu.async_copy(emb_hbm.at[idx_v.at[jp]], bufs[bp], gsems[bp])
      return carry

    lax.fori_loop(0, k_w // _ENB, body, 0)
    for b in range(_ENB):
      pltpu.make_async_copy(
          bufs[b], out_hbm.at[pl.ds(obase + (k_w - _ENB + b) * _CH, _CH)],
          wsems[b]).wait()

  return k


def _enc_body(x_ref, w_ref, b_ref, o_ref):
  o_ref[...] = (jnp.dot(x_ref[...], w_ref[0],
                        preferred_element_type=jnp.float32) + b_ref[0])


def _layer_body(t0, t1, t2, t3, m0, m1, m2, m3, deg_ref,
                ws_ref, wn_ref, b_ref, f_ref, o_ref):
  xfull = jnp.concatenate([t0[...], t1[...], t2[...], t3[...]], axis=1)
  msum = jnp.concatenate([m0[...], m1[...], m2[...], m3[...]], axis=1)
  mean = msum * (1.0 / jnp.maximum(deg_ref[...], 1.0))
  h = (jnp.dot(xfull, ws_ref[0], preferred_element_type=jnp.float32)
       + jnp.dot(mean, wn_ref[0], preferred_element_type=jnp.float32)
       + b_ref[0])
  o_ref[...] = jnp.where(f_ref[0, 0] > 0, jnp.maximum(h, 0.0), h)


def _final_body(t0, t1, t2, t3, w1_ref, b1_ref, w2_ref, b2_ref,
                emb_ref, no_ref):
  emb = jnp.concatenate([t0[...], t1[...], t2[...], t3[...]], axis=1)
  emb_ref[...] = emb
  h1 = jnp.maximum(
      jnp.dot(emb, w1_ref[...], preferred_element_type=jnp.float32)
      + b1_ref[...], 0.0)
  no_ref[...] = (jnp.dot(h1, w2_ref[...], preferred_element_type=jnp.float32)
                 + b2_ref[...])


def _edge_body(s_ref, d_ref, r_ref, w1s_ref, w1d_ref, w1r_ref, b1_ref,
               w2_ref, b2_ref, o_ref):
  bf = jnp.bfloat16
  h = (jnp.dot(s_ref[...].astype(bf), w1s_ref[...],
               preferred_element_type=jnp.float32)
       + jnp.dot(d_ref[...].astype(bf), w1d_ref[...],
                 preferred_element_type=jnp.float32)
       + jnp.dot(r_ref[...], w1r_ref[...], preferred_element_type=jnp.float32)
       + b1_ref[...])
  h = jnp.maximum(h, 0.0)
  o_ref[...] = (jnp.dot(h, w2_ref[...], preferred_element_type=jnp.float32)
                + b2_ref[...])


def _bcast(shape):
  return pl.BlockSpec(shape, lambda i: tuple(0 for _ in shape))


def _rows(blk, ncol):
  return pl.BlockSpec((blk, ncol), lambda i: (i, 0))


def _split_w(w128, bias):
  w = jnp.stack([w128[:, q * _PW:(q + 1) * _PW] for q in range(_NQ)])
  b = jnp.stack([bias[q * _PW:(q + 1) * _PW][None, :] for q in range(_NQ)])
  return w, b


def kernel(edge_index, node_static, edge_static, p_obs, q_obs, p_mask,
           q_mask, params):
  f32 = jnp.float32
  i32 = jnp.int32
  n = p_obs.shape[0]
  e = q_obs.shape[0]
  h = params["enc_W"].shape[1]
  blk_n = 1024
  n_pad = -(-n // blk_n) * blk_n
  grid_n = n_pad // blk_n

  ei = edge_index.astype(i32)
  eb = 2 * e
  k1 = -(-eb // (_NS * _CH))
  k1 = -(-k1 // 8) * 8
  eb_pad = _NS * k1 * _CH
  spread = jnp.arange(eb_pad - eb, dtype=i32)
  src_flat = jnp.concatenate([ei[0], ei[1], spread % n_pad])
  dst_flat = jnp.concatenate([ei[1], ei[0], n + spread % (n_pad - n)])
  src2d = src_flat.reshape(-1, _CH)
  src_all = jnp.concatenate([src2d + q * n_pad for q in range(_NQ)], axis=0)
  dst2d = dst_flat.reshape(-1, _CH)
  erng = jnp.arange(eb_pad, dtype=i32)
  dlane2d = ((dst_flat % 16) + 16 * (erng % _DREP)).reshape(-1, _CH)
  drow2d = (dst_flat // 16).reshape(-1, _CH)

  k2 = -(-e // (_NW * _CH))
  k2 = -(-k2 // 8) * 8
  e_pad = _NW * k2 * _CH
  pad_e = jnp.arange(e_pad - e, dtype=i32) % n_pad
  sidx = jnp.concatenate([ei[0], pad_e]).reshape(-1, _CH)
  didx = jnp.concatenate([ei[1], pad_e]).reshape(-1, _CH)

  feats = jnp.concatenate(
      [node_static, p_obs[:, None], p_mask[:, None].astype(f32)], axis=1)
  feats = jnp.pad(feats, ((0, n_pad - n), (0, 0)))
  zeros_tab = jnp.zeros((n_pad, _PW), f32)
  zeros_deg = jnp.zeros((n_pad // 16, 16), f32)
  eye_tab = jnp.tile(jnp.eye(16, dtype=f32), (_DREP, 1))
  enc_ws, enc_bs = _split_w(params["enc_W"], params["enc_b"])

  ws_all, wn_all, b_all = [], [], []
  for lyr in params["sage"]:
    ws_s, b_s = _split_w(lyr["Ws"], lyr["b"])
    wn_s, _ = _split_w(lyr["Wn"], jnp.zeros((h,), f32))
    ws_all.append(ws_s)
    wn_all.append(wn_s)
    b_all.append(b_s)
  ws_all = jnp.stack(ws_all)
  wn_all = jnp.stack(wn_all)
  b_all = jnp.stack(b_all)
  relu_fl = jnp.array([[[1.0]], [[1.0]], [[0.0]]], f32)

  nr, na = params["node_recon"], params["node_anom"]
  w1n = jnp.concatenate([nr["W1"], na["W1"]], axis=1)
  b1n = jnp.concatenate([nr["b1"], na["b1"]])[None, :]
  w2n = jnp.zeros((2 * h, 2), f32)
  w2n = w2n.at[:h, 0].set(nr["W2"][:, 0]).at[h:, 1].set(na["W2"][:, 0])
  b2n = jnp.concatenate([nr["b2"], na["b2"]])[None, :]

  er, ea = params["edge_recon"], params["edge_anom"]
  w1e = jnp.concatenate([er["W1"], ea["W1"]], axis=1)
  w1s, w1d, w1r = w1e[:h], w1e[h:2 * h], w1e[2 * h:]
  b1e = jnp.concatenate([er["b1"], ea["b1"]])[None, :]
  w2e = jnp.zeros((2 * h, 2), f32)
  w2e = w2e.at[:h, 0].set(er["W2"][:, 0]).at[h:, 1].set(ea["W2"][:, 0])
  b2e = jnp.concatenate([er["b2"], ea["b2"]])[None, :]
  rest = jnp.concatenate(
      [edge_static, q_obs[:, None], q_mask[:, None].astype(f32)], axis=1)
  w1s_bf = w1s.astype(jnp.bfloat16)
  w1d_bf = w1d.astype(jnp.bfloat16)

  dpart = _degree_call(n_pad, k1 // 2)(dlane2d, drow2d, eye_tab, zeros_deg)
  deg_col = (dpart[:n_pad // 16] + dpart[n_pad // 16:]).reshape(-1)[:, None]

  table = pl.pallas_call(
      _enc_body,
      grid=(_NQ * grid_n,),
      in_specs=[
          pl.BlockSpec((blk_n, h), lambda i: (i % grid_n, 0)),
          pl.BlockSpec((1, h, _PW), lambda i: (i // grid_n, 0, 0)),
          pl.BlockSpec((1, 1, _PW), lambda i: (i // grid_n, 0, 0)),
      ],
      out_specs=_rows(blk_n, _PW),
      out_shape=jax.ShapeDtypeStruct((_NQ * n_pad, _PW), f32),
  )(feats, enc_ws, enc_bs)

  seg = _seg_sum_call(n_pad, k1)
  qspec = [pl.BlockSpec((blk_n, _PW), lambda i, q=q: (i % grid_n + q * grid_n, 0))
           for q in range(_NQ)]
  dspec = pl.BlockSpec((blk_n, 1), lambda i: (i % grid_n, 0))

  def one_layer(tab, wts):
    ws_s, wn_s, b_s, fl = wts
    part = seg(src_all, dst2d, tab, zeros_tab)
    nxt = pl.pallas_call(
        _layer_body,
        grid=(_NQ * grid_n,),
        in_specs=qspec + qspec + [
            dspec,
            pl.BlockSpec((1, h, _PW), lambda i: (i // grid_n, 0, 0)),
            pl.BlockSpec((1, h, _PW), lambda i: (i // grid_n, 0, 0)),
            pl.BlockSpec((1, 1, _PW), lambda i: (i // grid_n, 0, 0)),
            _bcast((1, 1)),
        ],
        out_specs=_rows(blk_n, _PW),
        out_shape=jax.ShapeDtypeStruct((_NQ * n_pad, _PW), f32),
    )(tab, tab, tab, tab, part, part, part, part,
      deg_col, ws_s, wn_s, b_s, fl)
    return nxt, None

  table, _ = lax.scan(one_layer, table, (ws_all, wn_all, b_all, relu_fl))

  emb, node_out = pl.pallas_call(
      _final_body,
      grid=(grid_n,),
      in_specs=qspec + [
          _bcast((h, 2 * h)), _bcast((1, 2 * h)),
          _bcast((2 * h, 2)), _bcast((1, 2))],
      out_specs=[_rows(blk_n, h), _rows(blk_n, 2)],
      out_shape=[jax.ShapeDtypeStruct((n_pad, h), f32),
                 jax.ShapeDtypeStruct((n_pad, 2), f32)],
  )(table, table, table, table, w1n, b1n, w2n, b2n)

  idx_all = jnp.concatenate([sidx, didx], axis=0)
  sd_emb = _edge_gather_call(n_pad, e_pad)(emb, idx_all)
  blk_e = 1280
  off_e = e_pad // blk_e
  edge_out = pl.pallas_call(
      _edge_body,
      grid=(e // blk_e,),
      in_specs=[_rows(blk_e, h),
                pl.BlockSpec((blk_e, h), lambda i: (i + off_e, 0)),
                _rows(blk_e, 16),
                _bcast((h, 2 * h)), _bcast((h, 2 * h)), _bcast((16, 2 * h)),
                _bcast((1, 2 * h)), _bcast((2 * h, 2)), _bcast((1, 2))],
      out_specs=_rows(blk_e, 2),
      out_shape=jax.ShapeDtypeStruct((e, 2), f32),
  )(sd_emb, sd_emb, rest, w1s_bf, w1d_bf, w1r, b1e, w2e, b2e)

  return (node_out[:n, 0], edge_out[:, 0],
          node_out[:n, 1], edge_out[:, 1])


# speedup vs baseline: 1.0036x; 1.0036x over previous
import functools

import jax
import jax.numpy as jnp
from jax import lax
from jax.experimental import pallas as pl
from jax.experimental.pallas import tpu as pltpu
from jax.experimental.pallas import tpu_sc as plsc

_NC = 2
_NS = 16
_NW = _NC * _NS
_CH = 128
_PW = 32
_NQ = 4
_DREP = 256


def _sc_mesh():
  return plsc.VectorSubcoreMesh(core_axis_name="c", subcore_axis_name="s",
                                num_cores=_NC, num_subcores=_NS)


_NB = 4
_LK = 2


def _ring_seg(table_hbm, src_v, dst_v, acc, bufs, gsems, ssems, k):
  for j in range(_LK):
    pltpu.async_copy(table_hbm.at[src_v.at[j]], bufs[j], gsems[j])

  def body(g, carry):
    for b in range(_NB):
      j = g * _NB + b
      pltpu.make_async_copy(table_hbm.at[src_v.at[j]], bufs[b],
                            gsems[b]).wait()
      pltpu.async_copy(bufs[b], acc.at[dst_v.at[j]], ssems[b], add=True)
      jp = j + _LK
      bp = (b + _LK) % _NB

      @pl.when(jp < k)
      def _():
        @pl.when(jp >= _NB)
        def _():
          pltpu.make_async_copy(bufs[bp], acc.at[dst_v.at[jp - _NB]],
                                ssems[bp]).wait()
        pltpu.async_copy(table_hbm.at[src_v.at[jp]], bufs[bp], gsems[bp])
    return carry

  lax.fori_loop(0, k // _NB, body, 0)
  for b in range(_NB):
    pltpu.make_async_copy(bufs[b], acc.at[dst_v.at[k - _NB + b]],
                          ssems[b]).wait()


def _seg_sum_call(n_pad, k_per_tile):
  grp = n_pad // _NS
  rows = _NS * k_per_tile

  @functools.partial(
      pl.kernel,
      out_type=jax.ShapeDtypeStruct((_NQ * n_pad, _PW), jnp.float32),
      mesh=_sc_mesh(),
      compiler_params=pltpu.CompilerParams(use_tc_tiling_on_sc=False),
      scratch_types=(
          [pltpu.VMEM((k_per_tile, _CH), jnp.int32),
           pltpu.VMEM((k_per_tile, _CH), jnp.int32)]
          + [pltpu.VMEM((_CH, _PW), jnp.float32)] * _NB
          + [pltpu.VMEM_SHARED((n_pad, _PW), jnp.float32)]
          + [pltpu.SemaphoreType.DMA] * (2 * _NB)
      ),
  )
  def k(src_hbm, dst_hbm, table_hbm, zeros_hbm, out_hbm,
        src_v, dst_v, *rest):
    bufs = rest[:_NB]
    acc = rest[_NB]
    gsems = rest[_NB + 1:2 * _NB + 1]
    ssems = rest[2 * _NB + 1:]
    c = lax.axis_index("c")
    s = lax.axis_index("s")
    pltpu.sync_copy(dst_hbm.at[pl.ds(s * k_per_tile, k_per_tile)], dst_v)
    for p in range(2):
      q = 2 * c + p
      pltpu.sync_copy(zeros_hbm.at[pl.ds(s * grp, grp)],
                      acc.at[pl.ds(s * grp, grp)])
      pltpu.sync_copy(
          src_hbm.at[pl.ds(q * rows + s * k_per_tile, k_per_tile)], src_v)
      plsc.subcore_barrier()
      _ring_seg(table_hbm, src_v, dst_v, acc, bufs, gsems, ssems,
                k_per_tile)
      plsc.subcore_barrier()
      pltpu.sync_copy(acc.at[pl.ds(s * grp, grp)],
                      out_hbm.at[pl.ds(q * n_pad + s * grp, grp)])

  return k


def _degree_call(n_pad, k_per_tile):
  drows = n_pad // 16
  grp = drows // _NS

  @functools.partial(
      pl.kernel,
      out_type=jax.ShapeDtypeStruct((_NC * drows, 16), jnp.float32),
      mesh=_sc_mesh(),
      compiler_params=pltpu.CompilerParams(use_tc_tiling_on_sc=False),
      scratch_types=(
          [pltpu.VMEM((k_per_tile, _CH), jnp.int32),
           pltpu.VMEM((k_per_tile, _CH), jnp.int32)]
          + [pltpu.VMEM((_CH, 16), jnp.float32)] * _NB
          + [pltpu.VMEM_SHARED((drows, 16), jnp.float32)]
          + [pltpu.SemaphoreType.DMA] * (2 * _NB)
      ),
  )
  def k(lane_hbm, drow_hbm, eye_hbm, zeros_hbm, out_hbm,
        lane_v, drow_v, *rest):
    bufs = rest[:_NB]
    acc = rest[_NB]
    gsems = rest[_NB + 1:2 * _NB + 1]
    ssems = rest[2 * _NB + 1:]
    c = lax.axis_index("c")
    s = lax.axis_index("s")
    wid = s * _NC + c
    pltpu.sync_copy(zeros_hbm.at[pl.ds(s * grp, grp)],
                    acc.at[pl.ds(s * grp, grp)])
    base = wid * k_per_tile
    pltpu.sync_copy(lane_hbm.at[pl.ds(base, k_per_tile)], lane_v)
    pltpu.sync_copy(drow_hbm.at[pl.ds(base, k_per_tile)], drow_v)
    plsc.subcore_barrier()
    _ring_seg(eye_hbm, lane_v, drow_v, acc, bufs, gsems, ssems, k_per_tile)
    plsc.subcore_barrier()
    pltpu.sync_copy(acc.at[pl.ds(s * grp, grp)],
                    out_hbm.at[pl.ds(c * drows + s * grp, grp)])

  return k


_ENB = 4


def _edge_gather_call(n_pad, e_pad):
  k_w = e_pad // _CH // _NW
  lk = _ENB // 2

  @functools.partial(
      pl.kernel,
      out_type=jax.ShapeDtypeStruct((e_pad, 128), jnp.float32),
      mesh=_sc_mesh(),
      compiler_params=pltpu.CompilerParams(use_tc_tiling_on_sc=False),
      scratch_types=(
          [pltpu.VMEM((k_w, _CH), jnp.int32)]
          + [pltpu.VMEM((_CH, 128), jnp.float32)] * _ENB
          + [pltpu.SemaphoreType.DMA] * (2 * _ENB)
      ),
  )
  def k(emb_hbm, idx_hbm, out_hbm, idx_v, *rest):
    bufs = rest[:_ENB]
    gsems = rest[_ENB:2 * _ENB]
    wsems = rest[2 * _ENB:]
    c = lax.axis_index("c")
    s = lax.axis_index("s")
    wid = s * _NC + c
    pltpu.sync_copy(idx_hbm.at[pl.ds(wid * k_w, k_w)], idx_v)
    obase = wid * k_w * _CH

    for j in range(lk):
      pltpu.async_copy(emb_hbm.at[idx_v.at[j]], bufs[j], gsems[j])

    def body(g, carry):
      for b in range(_ENB):
        j = g * _ENB + b
        pltpu.make_async_copy(emb_hbm.at[idx_v.at[j]], bufs[b],
                              gsems[b]).wait()
        pltpu.async_copy(bufs[b], out_hbm.at[pl.ds(obase + j * _CH, _CH)],
                         wsems[b])
        jp = j + lk
        bp = (b + lk) % _ENB

        @pl.when(jp < k_w)
        def _():
          @pl.when(jp >= _ENB)
          def _():
            pltpu.make_async_copy(
                bufs[bp],
                out_hbm.at[pl.ds(obase + (jp - _ENB) * _CH, _CH)],
                wsems[bp]).wait()
          pltpu.async_copy(emb_hbm.at[idx_v.at[jp]], bufs[bp], gsems[bp])
      return carry

    lax.fori_loop(0, k_w // _ENB, body, 0)
    for b in range(_ENB):
      pltpu.make_async_copy(
          bufs[b], out_hbm.at[pl.ds(obase + (k_w - _ENB + b) * _CH, _CH)],
          wsems[b]).wait()

  return k


def _enc_body(x_ref, w_ref, b_ref, o_ref):
  o_ref[...] = (jnp.dot(x_ref[...], w_ref[0],
                        preferred_element_type=jnp.float32) + b_ref[0])


def _layer_body(t0, t1, t2, t3, m0, m1, m2, m3, deg_ref,
                ws_ref, wn_ref, b_ref, f_ref, o_ref):
  xfull = jnp.concatenate([t0[...], t1[...], t2[...], t3[...]], axis=1)
  msum = jnp.concatenate([m0[...], m1[...], m2[...], m3[...]], axis=1)
  mean = msum * (1.0 / jnp.maximum(deg_ref[...], 1.0))
  h = (jnp.dot(xfull, ws_ref[0], preferred_element_type=jnp.float32)
       + jnp.dot(mean, wn_ref[0], preferred_element_type=jnp.float32)
       + b_ref[0])
  o_ref[...] = jnp.where(f_ref[0, 0] > 0, jnp.maximum(h, 0.0), h)


def _final_body(t0, t1, t2, t3, w1_ref, b1_ref, w2_ref, b2_ref,
                emb_ref, no_ref):
  emb = jnp.concatenate([t0[...], t1[...], t2[...], t3[...]], axis=1)
  emb_ref[...] = emb
  h1 = jnp.maximum(
      jnp.dot(emb, w1_ref[...], preferred_element_type=jnp.float32)
      + b1_ref[...], 0.0)
  no_ref[...] = (jnp.dot(h1, w2_ref[...], preferred_element_type=jnp.float32)
                 + b2_ref[...])


def _edge_body(s_ref, d_ref, r_ref, w1s_ref, w1d_ref, w1r_ref, b1_ref,
               w2_ref, b2_ref, o_ref):
  bf = jnp.bfloat16
  h = (jnp.dot(s_ref[...].astype(bf), w1s_ref[...],
               preferred_element_type=jnp.float32)
       + jnp.dot(d_ref[...].astype(bf), w1d_ref[...],
                 preferred_element_type=jnp.float32)
       + jnp.dot(r_ref[...], w1r_ref[...], preferred_element_type=jnp.float32)
       + b1_ref[...])
  h = jnp.maximum(h, 0.0)
  o_ref[...] = (jnp.dot(h, w2_ref[...], preferred_element_type=jnp.float32)
                + b2_ref[...])


def _bcast(shape):
  return pl.BlockSpec(shape, lambda i: tuple(0 for _ in shape))


def _rows(blk, ncol):
  return pl.BlockSpec((blk, ncol), lambda i: (i, 0))


def _split_w(w128, bias):
  w = jnp.stack([w128[:, q * _PW:(q + 1) * _PW] for q in range(_NQ)])
  b = jnp.stack([bias[q * _PW:(q + 1) * _PW][None, :] for q in range(_NQ)])
  return w, b


def kernel(edge_index, node_static, edge_static, p_obs, q_obs, p_mask,
           q_mask, params):
  f32 = jnp.float32
  i32 = jnp.int32
  n = p_obs.shape[0]
  e = q_obs.shape[0]
  h = params["enc_W"].shape[1]
  blk_n = 1024
  n_pad = -(-n // blk_n) * blk_n
  grid_n = n_pad // blk_n

  ei = edge_index.astype(i32)
  eb = 2 * e
  k1 = -(-eb // (_NS * _CH))
  k1 = -(-k1 // 8) * 8
  eb_pad = _NS * k1 * _CH
  spread = jnp.arange(eb_pad - eb, dtype=i32)
  src_flat = jnp.concatenate([ei[0], ei[1], spread % n_pad])
  dst_flat = jnp.concatenate([ei[1], ei[0], n + spread % (n_pad - n)])
  src2d = src_flat.reshape(-1, _CH)
  src_all = jnp.concatenate([src2d + q * n_pad for q in range(_NQ)], axis=0)
  dst2d = dst_flat.reshape(-1, _CH)
  erng = jnp.arange(eb_pad, dtype=i32)
  dlane2d = ((dst_flat % 16) + 16 * (erng % _DREP)).reshape(-1, _CH)
  drow2d = (dst_flat // 16).reshape(-1, _CH)

  k2 = -(-e // (_NW * _CH))
  k2 = -(-k2 // 8) * 8
  e_pad = _NW * k2 * _CH
  pad_e = jnp.arange(e_pad - e, dtype=i32) % n_pad
  sidx = jnp.concatenate([ei[0], pad_e]).reshape(-1, _CH)
  didx = jnp.concatenate([ei[1], pad_e]).reshape(-1, _CH)

  feats = jnp.concatenate(
      [node_static, p_obs[:, None], p_mask[:, None].astype(f32)], axis=1)
  feats = jnp.pad(feats, ((0, n_pad - n), (0, 0)))
  zeros_tab = jnp.zeros((n_pad, _PW), f32)
  zeros_deg = jnp.zeros((n_pad // 16, 16), f32)
  eye_tab = jnp.tile(jnp.eye(16, dtype=f32), (_DREP, 1))
  enc_ws, enc_bs = _split_w(params["enc_W"], params["enc_b"])

  ws_all, wn_all, b_all = [], [], []
  for lyr in params["sage"]:
    ws_s, b_s = _split_w(lyr["Ws"], lyr["b"])
    wn_s, _ = _split_w(lyr["Wn"], jnp.zeros((h,), f32))
    ws_all.append(ws_s)
    wn_all.append(wn_s)
    b_all.append(b_s)
  ws_all = jnp.stack(ws_all)
  wn_all = jnp.stack(wn_all)
  b_all = jnp.stack(b_all)
  relu_fl = jnp.array([[[1.0]], [[1.0]], [[0.0]]], f32)

  nr, na = params["node_recon"], params["node_anom"]
  w1n = jnp.concatenate([nr["W1"], na["W1"]], axis=1)
  b1n = jnp.concatenate([nr["b1"], na["b1"]])[None, :]
  w2n = jnp.zeros((2 * h, 2), f32)
  w2n = w2n.at[:h, 0].set(nr["W2"][:, 0]).at[h:, 1].set(na["W2"][:, 0])
  b2n = jnp.concatenate([nr["b2"], na["b2"]])[None, :]

  er, ea = params["edge_recon"], params["edge_anom"]
  w1e = jnp.concatenate([er["W1"], ea["W1"]], axis=1)
  w1s, w1d, w1r = w1e[:h], w1e[h:2 * h], w1e[2 * h:]
  b1e = jnp.concatenate([er["b1"], ea["b1"]])[None, :]
  w2e = jnp.zeros((2 * h, 2), f32)
  w2e = w2e.at[:h, 0].set(er["W2"][:, 0]).at[h:, 1].set(ea["W2"][:, 0])
  b2e = jnp.concatenate([er["b2"], ea["b2"]])[None, :]
  rest = jnp.concatenate(
      [edge_static, q_obs[:, None], q_mask[:, None].astype(f32)], axis=1)
  w1s_bf = w1s.astype(jnp.bfloat16)
  w1d_bf = w1d.astype(jnp.bfloat16)

  dpart = _degree_call(n_pad, k1 // 2)(dlane2d, drow2d, eye_tab, zeros_deg)
  deg_col = (dpart[:n_pad // 16] + dpart[n_pad // 16:]).reshape(-1)[:, None]

  table = pl.pallas_call(
      _enc_body,
      grid=(_NQ * grid_n,),
      in_specs=[
          pl.BlockSpec((blk_n, h), lambda i: (i % grid_n, 0)),
          pl.BlockSpec((1, h, _PW), lambda i: (i // grid_n, 0, 0)),
          pl.BlockSpec((1, 1, _PW), lambda i: (i // grid_n, 0, 0)),
      ],
      out_specs=_rows(blk_n, _PW),
      out_shape=jax.ShapeDtypeStruct((_NQ * n_pad, _PW), f32),
  )(feats, enc_ws, enc_bs)

  seg = _seg_sum_call(n_pad, k1)
  qspec = [pl.BlockSpec((blk_n, _PW), lambda i, q=q: (i % grid_n + q * grid_n, 0))
           for q in range(_NQ)]
  dspec = pl.BlockSpec((blk_n, 1), lambda i: (i % grid_n, 0))

  def one_layer(tab, wts):
    ws_s, wn_s, b_s, fl = wts
    part = seg(src_all, dst2d, tab, zeros_tab)
    nxt = pl.pallas_call(
        _layer_body,
        grid=(_NQ * grid_n,),
        in_specs=qspec + qspec + [
            dspec,
            pl.BlockSpec((1, h, _PW), lambda i: (i // grid_n, 0, 0)),
            pl.BlockSpec((1, h, _PW), lambda i: (i // grid_n, 0, 0)),
            pl.BlockSpec((1, 1, _PW), lambda i: (i // grid_n, 0, 0)),
            _bcast((1, 1)),
        ],
        out_specs=_rows(blk_n, _PW),
        out_shape=jax.ShapeDtypeStruct((_NQ * n_pad, _PW), f32),
    )(tab, tab, tab, tab, part, part, part, part,
      deg_col, ws_s, wn_s, b_s, fl)
    return nxt, None

  table, _ = lax.scan(one_layer, table, (ws_all, wn_all, b_all, relu_fl))

  emb, node_out = pl.pallas_call(
      _final_body,
      grid=(grid_n,),
      in_specs=qspec + [
          _bcast((h, 2 * h)), _bcast((1, 2 * h)),
          _bcast((2 * h, 2)), _bcast((1, 2))],
      out_specs=[_rows(blk_n, h), _rows(blk_n, 2)],
      out_shape=[jax.ShapeDtypeStruct((n_pad, h), f32),
                 jax.ShapeDtypeStruct((n_pad, 2), f32)],
  )(table, table, table, table, w1n, b1n, w2n, b2n)

  egather = _edge_gather_call(n_pad, e_pad)
  s_emb = egather(emb, sidx)
  d_emb = egather(emb, didx)
  blk_e = 1280
  edge_out = pl.pallas_call(
      _edge_body,
      grid=(e // blk_e,),
      in_specs=[_rows(blk_e, h), _rows(blk_e, h), _rows(blk_e, 16),
                _bcast((h, 2 * h)), _bcast((h, 2 * h)), _bcast((16, 2 * h)),
                _bcast((1, 2 * h)), _bcast((2 * h, 2)), _bcast((1, 2))],
      out_specs=_rows(blk_e, 2),
      out_shape=jax.ShapeDtypeStruct((e, 2), f32),
  )(s_emb, d_emb, rest, w1s_bf, w1d_bf, w1r, b1e, w2e, b2e)

  return (node_out[:n, 0], edge_out[:, 0],
          node_out[:n, 1], edge_out[:, 1])


# speedup vs baseline: 1.2394x; 1.2349x over previous
import functools

import jax
import jax.numpy as jnp
from jax import lax
from jax.experimental import pallas as pl
from jax.experimental.pallas import tpu as pltpu
from jax.experimental.pallas import tpu_sc as plsc

_NC = 2
_NS = 16
_NW = _NC * _NS
_CH = 128
_PW = 32
_NQ = 4
_DREP = 256


def _sc_mesh():
  return plsc.VectorSubcoreMesh(core_axis_name="c", subcore_axis_name="s",
                                num_cores=_NC, num_subcores=_NS)


_NB = 4
_LK = 2


def _ring_seg(table_hbm, src_v, dst_v, acc, bufs, gsems, ssems, k):
  for j in range(_LK):
    pltpu.async_copy(table_hbm.at[src_v.at[j]], bufs[j], gsems[j])

  def body(g, carry):
    for b in range(_NB):
      j = g * _NB + b
      pltpu.make_async_copy(table_hbm.at[src_v.at[j]], bufs[b],
                            gsems[b]).wait()
      pltpu.async_copy(bufs[b], acc.at[dst_v.at[j]], ssems[b], add=True)
      jp = j + _LK
      bp = (b + _LK) % _NB

      @pl.when(jp < k)
      def _():
        @pl.when(jp >= _NB)
        def _():
          pltpu.make_async_copy(bufs[bp], acc.at[dst_v.at[jp - _NB]],
                                ssems[bp]).wait()
        pltpu.async_copy(table_hbm.at[src_v.at[jp]], bufs[bp], gsems[bp])
    return carry

  lax.fori_loop(0, k // _NB, body, 0)
  for b in range(_NB):
    pltpu.make_async_copy(bufs[b], acc.at[dst_v.at[k - _NB + b]],
                          ssems[b]).wait()


def _seg_sum_call(n_pad, k_per_tile):
  grp = n_pad // _NS
  rows = _NS * k_per_tile

  @functools.partial(
      pl.kernel,
      out_type=jax.ShapeDtypeStruct((n_pad, _NQ * _PW), jnp.float32),
      mesh=_sc_mesh(),
      compiler_params=pltpu.CompilerParams(use_tc_tiling_on_sc=False),
      scratch_types=(
          [pltpu.VMEM((k_per_tile, _CH), jnp.int32),
           pltpu.VMEM((k_per_tile, _CH), jnp.int32)]
          + [pltpu.VMEM((_CH, _PW), jnp.float32)] * _NB
          + [pltpu.VMEM_SHARED((n_pad, _PW), jnp.float32)]
          + [pltpu.SemaphoreType.DMA] * (2 * _NB)
      ),
  )
  def k(src_hbm, dst_hbm, table_hbm, zeros_hbm, out_hbm,
        src_v, dst_v, *rest):
    bufs = rest[:_NB]
    acc = rest[_NB]
    gsems = rest[_NB + 1:2 * _NB + 1]
    ssems = rest[2 * _NB + 1:]
    c = lax.axis_index("c")
    s = lax.axis_index("s")
    pltpu.sync_copy(dst_hbm.at[pl.ds(s * k_per_tile, k_per_tile)], dst_v)
    for p in range(2):
      q = 2 * c + p
      pltpu.sync_copy(zeros_hbm.at[pl.ds(s * grp, grp)],
                      acc.at[pl.ds(s * grp, grp)])
      pltpu.sync_copy(
          src_hbm.at[pl.ds(q * rows + s * k_per_tile, k_per_tile)], src_v)
      plsc.subcore_barrier()
      _ring_seg(table_hbm, src_v, dst_v, acc, bufs, gsems, ssems,
                k_per_tile)
      plsc.subcore_barrier()
      pltpu.sync_copy(acc.at[pl.ds(s * grp, grp)],
                      out_hbm.at[pl.ds(s * grp, grp), pl.ds(q * _PW, _PW)])

  return k


def _degree_call(n_pad, k_per_tile):
  drows = n_pad // 16
  grp = drows // _NS

  @functools.partial(
      pl.kernel,
      out_type=jax.ShapeDtypeStruct((_NC * drows, 16), jnp.float32),
      mesh=_sc_mesh(),
      compiler_params=pltpu.CompilerParams(use_tc_tiling_on_sc=False),
      scratch_types=(
          [pltpu.VMEM((k_per_tile, _CH), jnp.int32),
           pltpu.VMEM((k_per_tile, _CH), jnp.int32)]
          + [pltpu.VMEM((_CH, 16), jnp.float32)] * _NB
          + [pltpu.VMEM_SHARED((drows, 16), jnp.float32)]
          + [pltpu.SemaphoreType.DMA] * (2 * _NB)
      ),
  )
  def k(lane_hbm, drow_hbm, eye_hbm, zeros_hbm, out_hbm,
        lane_v, drow_v, *rest):
    bufs = rest[:_NB]
    acc = rest[_NB]
    gsems = rest[_NB + 1:2 * _NB + 1]
    ssems = rest[2 * _NB + 1:]
    c = lax.axis_index("c")
    s = lax.axis_index("s")
    wid = s * _NC + c
    pltpu.sync_copy(zeros_hbm.at[pl.ds(s * grp, grp)],
                    acc.at[pl.ds(s * grp, grp)])
    base = wid * k_per_tile
    pltpu.sync_copy(lane_hbm.at[pl.ds(base, k_per_tile)], lane_v)
    pltpu.sync_copy(drow_hbm.at[pl.ds(base, k_per_tile)], drow_v)
    plsc.subcore_barrier()
    _ring_seg(eye_hbm, lane_v, drow_v, acc, bufs, gsems, ssems, k_per_tile)
    plsc.subcore_barrier()
    pltpu.sync_copy(acc.at[pl.ds(s * grp, grp)],
                    out_hbm.at[pl.ds(c * drows + s * grp, grp)])

  return k


_ENB = 4


def _edge_gather_call(n_pad, e_pad):
  k_w = e_pad // _CH // _NW
  lk = _ENB // 2

  @functools.partial(
      pl.kernel,
      out_type=jax.ShapeDtypeStruct((e_pad, 128), jnp.float32),
      mesh=_sc_mesh(),
      compiler_params=pltpu.CompilerParams(use_tc_tiling_on_sc=False),
      scratch_types=(
          [pltpu.VMEM((k_w, _CH), jnp.int32)]
          + [pltpu.VMEM((_CH, 128), jnp.float32)] * _ENB
          + [pltpu.SemaphoreType.DMA] * (2 * _ENB)
      ),
  )
  def k(emb_hbm, idx_hbm, out_hbm, idx_v, *rest):
    bufs = rest[:_ENB]
    gsems = rest[_ENB:2 * _ENB]
    wsems = rest[2 * _ENB:]
    c = lax.axis_index("c")
    s = lax.axis_index("s")
    wid = s * _NC + c
    pltpu.sync_copy(idx_hbm.at[pl.ds(wid * k_w, k_w)], idx_v)
    obase = wid * k_w * _CH

    for j in range(lk):
      pltpu.async_copy(emb_hbm.at[idx_v.at[j]], bufs[j], gsems[j])

    def body(g, carry):
      for b in range(_ENB):
        j = g * _ENB + b
        pltpu.make_async_copy(emb_hbm.at[idx_v.at[j]], bufs[b],
                              gsems[b]).wait()
        pltpu.async_copy(bufs[b], out_hbm.at[pl.ds(obase + j * _CH, _CH)],
                         wsems[b])
        jp = j + lk
        bp = (b + lk) % _ENB

        @pl.when(jp < k_w)
        def _():
          @pl.when(jp >= _ENB)
          def _():
            pltpu.make_async_copy(
                bufs[bp],
                out_hbm.at[pl.ds(obase + (jp - _ENB) * _CH, _CH)],
                wsems[bp]).wait()
          pltpu.async_copy(emb_hbm.at[idx_v.at[jp]], bufs[bp], gsems[bp])
      return carry

    lax.fori_loop(0, k_w // _ENB, body, 0)
    for b in range(_ENB):
      pltpu.make_async_copy(
          bufs[b], out_hbm.at[pl.ds(obase + (k_w - _ENB + b) * _CH, _CH)],
          wsems[b]).wait()

  return k


def _enc_body(x_ref, w_ref, b_ref, o_ref):
  o_ref[...] = (jnp.dot(x_ref[...], w_ref[...],
                        preferred_element_type=jnp.float32) + b_ref[...])


def _layer_body(t_ref, m_ref, deg_ref, ws_ref, wn_ref, b_ref, f_ref,
                o_ref):
  mean = m_ref[...] * (1.0 / jnp.maximum(deg_ref[...], 1.0))
  h = (jnp.dot(t_ref[...], ws_ref[...], preferred_element_type=jnp.float32)
       + jnp.dot(mean, wn_ref[...], preferred_element_type=jnp.float32)
       + b_ref[...])
  o_ref[...] = jnp.where(f_ref[0, 0] > 0, jnp.maximum(h, 0.0), h)


def _final_body(t_ref, w1_ref, b1_ref, w2_ref, b2_ref, no_ref):
  h1 = jnp.maximum(
      jnp.dot(t_ref[...], w1_ref[...], preferred_element_type=jnp.float32)
      + b1_ref[...], 0.0)
  no_ref[...] = (jnp.dot(h1, w2_ref[...], preferred_element_type=jnp.float32)
                 + b2_ref[...])


def _edge_body(s_ref, d_ref, r_ref, w1s_ref, w1d_ref, w1r_ref, b1_ref,
               w2_ref, b2_ref, o_ref):
  bf = jnp.bfloat16
  h = (jnp.dot(s_ref[...].astype(bf), w1s_ref[...],
               preferred_element_type=jnp.float32)
       + jnp.dot(d_ref[...].astype(bf), w1d_ref[...],
                 preferred_element_type=jnp.float32)
       + jnp.dot(r_ref[...], w1r_ref[...], preferred_element_type=jnp.float32)
       + b1_ref[...])
  h = jnp.maximum(h, 0.0)
  o_ref[...] = (jnp.dot(h, w2_ref[...], preferred_element_type=jnp.float32)
                + b2_ref[...])


def _bcast(shape):
  return pl.BlockSpec(shape, lambda i: tuple(0 for _ in shape))


def _rows(blk, ncol):
  return pl.BlockSpec((blk, ncol), lambda i: (i, 0))


def kernel(edge_index, node_static, edge_static, p_obs, q_obs, p_mask,
           q_mask, params):
  f32 = jnp.float32
  i32 = jnp.int32
  n = p_obs.shape[0]
  e = q_obs.shape[0]
  h = params["enc_W"].shape[1]
  blk_n = 1024
  n_pad = -(-n // blk_n) * blk_n
  grid_n = n_pad // blk_n

  ei = edge_index.astype(i32)
  eb = 2 * e
  k1 = -(-eb // (_NS * _CH))
  k1 = -(-k1 // 8) * 8
  eb_pad = _NS * k1 * _CH
  spread = jnp.arange(eb_pad - eb, dtype=i32)
  src_flat = jnp.concatenate([ei[0], ei[1], spread % n_pad])
  dst_flat = jnp.concatenate([ei[1], ei[0], n + spread % (n_pad - n)])
  src2d = src_flat.reshape(-1, _CH)
  src_all = jnp.concatenate([_NQ * src2d + q for q in range(_NQ)], axis=0)
  dst2d = dst_flat.reshape(-1, _CH)
  erng = jnp.arange(eb_pad, dtype=i32)
  dlane2d = ((dst_flat % 16) + 16 * (erng % _DREP)).reshape(-1, _CH)
  drow2d = (dst_flat // 16).reshape(-1, _CH)

  k2 = -(-e // (_NW * _CH))
  k2 = -(-k2 // 8) * 8
  e_pad = _NW * k2 * _CH
  pad_e = jnp.arange(e_pad - e, dtype=i32) % n_pad
  sidx = jnp.concatenate([ei[0], pad_e]).reshape(-1, _CH)
  didx = jnp.concatenate([ei[1], pad_e]).reshape(-1, _CH)

  feats = jnp.concatenate(
      [node_static, p_obs[:, None], p_mask[:, None].astype(f32)], axis=1)
  feats = jnp.pad(feats, ((0, n_pad - n), (0, 0)))
  zeros_tab = jnp.zeros((n_pad, _PW), f32)
  zeros_deg = jnp.zeros((n_pad // 16, 16), f32)
  eye_tab = jnp.tile(jnp.eye(16, dtype=f32), (_DREP, 1))

  ws_all = jnp.stack([lyr["Ws"] for lyr in params["sage"]])
  wn_all = jnp.stack([lyr["Wn"] for lyr in params["sage"]])
  b_all = jnp.stack([lyr["b"][None, :] for lyr in params["sage"]])
  relu_fl = jnp.array([[[1.0]], [[1.0]], [[0.0]]], f32)

  nr, na = params["node_recon"], params["node_anom"]
  w1n = jnp.concatenate([nr["W1"], na["W1"]], axis=1)
  b1n = jnp.concatenate([nr["b1"], na["b1"]])[None, :]
  w2n = jnp.zeros((2 * h, 2), f32)
  w2n = w2n.at[:h, 0].set(nr["W2"][:, 0]).at[h:, 1].set(na["W2"][:, 0])
  b2n = jnp.concatenate([nr["b2"], na["b2"]])[None, :]

  er, ea = params["edge_recon"], params["edge_anom"]
  w1e = jnp.concatenate([er["W1"], ea["W1"]], axis=1)
  w1s, w1d, w1r = w1e[:h], w1e[h:2 * h], w1e[2 * h:]
  b1e = jnp.concatenate([er["b1"], ea["b1"]])[None, :]
  w2e = jnp.zeros((2 * h, 2), f32)
  w2e = w2e.at[:h, 0].set(er["W2"][:, 0]).at[h:, 1].set(ea["W2"][:, 0])
  b2e = jnp.concatenate([er["b2"], ea["b2"]])[None, :]
  rest = jnp.concatenate(
      [edge_static, q_obs[:, None], q_mask[:, None].astype(f32)], axis=1)
  w1s_bf = w1s.astype(jnp.bfloat16)
  w1d_bf = w1d.astype(jnp.bfloat16)

  dpart = _degree_call(n_pad, k1 // 2)(dlane2d, drow2d, eye_tab, zeros_deg)
  deg_col = (dpart[:n_pad // 16] + dpart[n_pad // 16:]).reshape(-1)[:, None]

  table = pl.pallas_call(
      _enc_body,
      grid=(grid_n,),
      in_specs=[_rows(blk_n, h), _bcast((h, h)), _bcast((1, h))],
      out_specs=_rows(blk_n, h),
      out_shape=jax.ShapeDtypeStruct((n_pad, h), f32),
  )(feats, params["enc_W"], params["enc_b"][None, :])

  seg = _seg_sum_call(n_pad, k1)

  def one_layer(tab, wts):
    ws_l, wn_l, b_l, fl = wts
    part = seg(src_all, dst2d, tab.reshape(_NQ * n_pad, _PW), zeros_tab)
    nxt = pl.pallas_call(
        _layer_body,
        grid=(grid_n,),
        in_specs=[_rows(blk_n, h), _rows(blk_n, h), _rows(blk_n, 1),
                  _bcast((h, h)), _bcast((h, h)), _bcast((1, h)),
                  _bcast((1, 1))],
        out_specs=_rows(blk_n, h),
        out_shape=jax.ShapeDtypeStruct((n_pad, h), f32),
    )(tab, part, deg_col, ws_l, wn_l, b_l, fl)
    return nxt, None

  table, _ = lax.scan(one_layer, table, (ws_all, wn_all, b_all, relu_fl))
  emb = table

  node_out = pl.pallas_call(
      _final_body,
      grid=(grid_n,),
      in_specs=[_rows(blk_n, h),
                _bcast((h, 2 * h)), _bcast((1, 2 * h)),
                _bcast((2 * h, 2)), _bcast((1, 2))],
      out_specs=_rows(blk_n, 2),
      out_shape=jax.ShapeDtypeStruct((n_pad, 2), f32),
  )(table, w1n, b1n, w2n, b2n)

  egather = _edge_gather_call(n_pad, e_pad)
  s_emb = egather(emb, sidx)
  d_emb = egather(emb, didx)
  blk_e = 2000
  edge_out = pl.pallas_call(
      _edge_body,
      grid=(e // blk_e,),
      in_specs=[_rows(blk_e, h), _rows(blk_e, h), _rows(blk_e, 16),
                _bcast((h, 2 * h)), _bcast((h, 2 * h)), _bcast((16, 2 * h)),
                _bcast((1, 2 * h)), _bcast((2 * h, 2)), _bcast((1, 2))],
      out_specs=_rows(blk_e, 2),
      out_shape=jax.ShapeDtypeStruct((e, 2), f32),
  )(s_emb, d_emb, rest, w1s_bf, w1d_bf, w1r, b1e, w2e, b2e)

  return (node_out[:n, 0], edge_out[:, 0],
          node_out[:n, 1], edge_out[:, 1])
